# Initial kernel scaffold; baseline (speedup 1.0000x reference)
import jax
import jax.numpy as jnp
from jax.experimental import pallas as pl


def kernel(x, adj_t, W1, a_src1, a_dst1, b1, W2, a_src2, a_dst2, b2, W3, a_src3, a_dst3, b3):
    raise NotImplementedError("write your pallas kernel here")


# speedup vs baseline: 26.2891x; 26.2891x over previous
import functools

import jax
import jax.numpy as jnp
from jax import lax
from jax.experimental import pallas as pl
from jax.experimental.pallas import tpu as pltpu
from jax.experimental.pallas import tpu_sc as plsc

N = 10000
EDGES = 320000
F = 128

NC, NS = 2, 16
NW = NC * NS
EW = EDGES // NW
K = 80
NB = EW // K
RPT = 624
RTAIL = N - NS * RPT
DPAD = 640 * NS

BN = 1000


def _pre_body(x_ref, w_ref, a2_ref, h_ref, al2_ref, es_ref):
    h = jnp.dot(x_ref[...], w_ref[...], preferred_element_type=jnp.float32)
    h_ref[...] = h
    al2 = jnp.dot(h, a2_ref[...], preferred_element_type=jnp.float32)
    al2_ref[...] = al2
    e = al2[:, :1] + al2[:, 1:2]
    es_ref[...] = jnp.where(e >= 0, e, 0.2 * e)


def _pre(x, W, a2):
    return pl.pallas_call(
        _pre_body,
        grid=(N // BN,),
        in_specs=[pl.BlockSpec((BN, F), lambda i: (i, 0)),
                  pl.BlockSpec((F, F), lambda i: (0, 0)),
                  pl.BlockSpec((F, 2), lambda i: (0, 0))],
        out_specs=[pl.BlockSpec((BN, F), lambda i: (i, 0)),
                   pl.BlockSpec((BN, 2), lambda i: (i, 0)),
                   pl.BlockSpec((BN, 1), lambda i: (i, 0))],
        out_shape=[jax.ShapeDtypeStruct((N, F), jnp.float32),
                   jax.ShapeDtypeStruct((N, 2), jnp.float32),
                   jax.ShapeDtypeStruct((N, 1), jnp.float32)],
    )(x, W, a2)


def _mid_body(p0_ref, p1_ref, d0_ref, d1_ref, hp_ref, b_ref, w_ref, a2_ref,
              hn_ref, al2_ref, es_ref):
    acc = p0_ref[...] + p1_ref[...] + hp_ref[...]
    inv = 1.0 / (d0_ref[...] + d1_ref[...] + 1.0 + 1e-16)
    o = acc * inv + b_ref[...]
    act = jnp.where(o > 0, o, jnp.exp(o) - 1.0)
    hn = jnp.dot(act, w_ref[...], preferred_element_type=jnp.float32)
    hn_ref[...] = hn
    al2 = jnp.dot(hn, a2_ref[...], preferred_element_type=jnp.float32)
    al2_ref[...] = al2
    e = al2[:, :1] + al2[:, 1:2]
    es_ref[...] = jnp.where(e >= 0, e, 0.2 * e)


def _mid(p0, p1, d0, d1, hp, b, W, a2):
    return pl.pallas_call(
        _mid_body,
        grid=(N // BN,),
        in_specs=[pl.BlockSpec((BN, F), lambda i: (i, 0)),
                  pl.BlockSpec((BN, F), lambda i: (i, 0)),
                  pl.BlockSpec((BN, 1), lambda i: (i, 0)),
                  pl.BlockSpec((BN, 1), lambda i: (i, 0)),
                  pl.BlockSpec((BN, F), lambda i: (i, 0)),
                  pl.BlockSpec((1, F), lambda i: (0, 0)),
                  pl.BlockSpec((F, F), lambda i: (0, 0)),
                  pl.BlockSpec((F, 2), lambda i: (0, 0))],
        out_specs=[pl.BlockSpec((BN, F), lambda i: (i, 0)),
                   pl.BlockSpec((BN, 2), lambda i: (i, 0)),
                   pl.BlockSpec((BN, 1), lambda i: (i, 0))],
        out_shape=[jax.ShapeDtypeStruct((N, F), jnp.float32),
                   jax.ShapeDtypeStruct((N, 2), jnp.float32),
                   jax.ShapeDtypeStruct((N, 1), jnp.float32)],
    )(p0, p1, d0, d1, hp, b, W, a2)


def _fin_body(p0_ref, p1_ref, d0_ref, d1_ref, hp_ref, b_ref, out_ref):
    acc = p0_ref[...] + p1_ref[...] + hp_ref[...]
    inv = 1.0 / (d0_ref[...] + d1_ref[...] + 1.0 + 1e-16)
    o = acc * inv + b_ref[...]
    m = jnp.max(o, axis=-1, keepdims=True)
    z = o - m
    out_ref[...] = z - jnp.log(jnp.sum(jnp.exp(z), axis=-1, keepdims=True))


def _fin(p0, p1, d0, d1, hp, b):
    return pl.pallas_call(
        _fin_body,
        grid=(N // BN,),
        in_specs=[pl.BlockSpec((BN, F), lambda i: (i, 0)),
                  pl.BlockSpec((BN, F), lambda i: (i, 0)),
                  pl.BlockSpec((BN, 1), lambda i: (i, 0)),
                  pl.BlockSpec((BN, 1), lambda i: (i, 0)),
                  pl.BlockSpec((BN, F), lambda i: (i, 0)),
                  pl.BlockSpec((1, F), lambda i: (0, 0))],
        out_specs=pl.BlockSpec((BN, F), lambda i: (i, 0)),
        out_shape=jax.ShapeDtypeStruct((N, F), jnp.float32),
    )(p0, p1, d0, d1, hp, b)


_MESH = plsc.VectorSubcoreMesh(core_axis_name="c", subcore_axis_name="s")


@functools.partial(
    pl.kernel,
    out_type=(jax.ShapeDtypeStruct((N, F), jnp.float32),
              jax.ShapeDtypeStruct((N, F), jnp.float32),
              jax.ShapeDtypeStruct((DPAD,), jnp.float32),
              jax.ShapeDtypeStruct((DPAD,), jnp.float32)),
    mesh=_MESH,
    compiler_params=pltpu.CompilerParams(needs_layout_passes=False),
    scratch_types=[
        pltpu.VMEM_SHARED((N, F), jnp.float32),
        pltpu.VMEM_SHARED((DPAD,), jnp.float32),
        pltpu.VMEM((N,), jnp.float32),
        pltpu.VMEM((N,), jnp.float32),
        pltpu.VMEM((N,), jnp.float32),
        pltpu.VMEM((K,), jnp.int32),
        pltpu.VMEM((K,), jnp.int32),
        pltpu.VMEM((K,), jnp.float32),
        pltpu.VMEM((K, F), jnp.float32),
        pltpu.VMEM((640,), jnp.float32),
        pltpu.SemaphoreType.DMA,
    ],
)
def _edge_pass(sidx, didx, h, als, ald, esv,
               p0, p1, dn0, dn1,
               accum, dnacc, as_l, ad_l, es_l, sb, db, exb, rows, zb, sem):
    cid = lax.axis_index("c")
    sid = lax.axis_index("s")
    wid = cid * NS + sid

    pltpu.sync_copy(als, as_l)
    pltpu.sync_copy(ald, ad_l)
    pltpu.sync_copy(esv, es_l)

    z16 = jnp.zeros((16,), jnp.float32)

    def _z1(i, c):
        zb[pl.ds(i * 16, 16)] = z16
        return c
    lax.fori_loop(0, 640 // 16, _z1, 0)

    def _zr(k, c):
        for g in range(F // 16):
            rows[k, pl.ds(g * 16, 16)] = z16
        return c
    lax.fori_loop(0, K, _zr, 0)

    pltpu.sync_copy(zb, dnacc.at[pl.ds(sid * 640, 640)])
    r0 = sid * RPT
    nfull = RPT // K
    rem = RPT - nfull * K

    def _za(i, c):
        pltpu.sync_copy(rows, accum.at[pl.ds(r0 + i * K, K)])
        return c
    lax.fori_loop(0, nfull, _za, 0)
    pltpu.sync_copy(rows.at[pl.ds(0, rem)], accum.at[pl.ds(r0 + nfull * K, rem)])

    @pl.when(sid == NS - 1)
    def _():
        pltpu.sync_copy(rows.at[pl.ds(0, RTAIL)],
                        accum.at[pl.ds(NS * RPT, RTAIL)])
    plsc.subcore_barrier()

    base = wid * EW

    def _blk(i, c):
        off = base + i * K
        pltpu.sync_copy(sidx.at[pl.ds(off, K)], sb)
        pltpu.sync_copy(didx.at[pl.ds(off, K)], db)
        cp = pltpu.async_copy(h.at[sb], rows, sem)

        def _grp(j, c2):
            s16 = sb[pl.ds(j * 16, 16)]
            d16 = db[pl.ds(j * 16, 16)]
            a = plsc.load_gather(as_l, [s16]) + plsc.load_gather(ad_l, [d16])
            e = jnp.where(a >= 0, a, 0.2 * a)
            exb[pl.ds(j * 16, 16)] = jnp.exp(e - plsc.load_gather(es_l, [d16]))
            return c2
        lax.fori_loop(0, K // 16, _grp, 0)
        cp.wait()

        def _srow(j, c2):
            ex16 = exb[pl.ds(j * 16, 16)]
            bk = j * 16
            for l in range(16):
                s = ex16[l]
                for g in range(F // 16):
                    rows[bk + l, pl.ds(g * 16, 16)] = rows[bk + l, pl.ds(g * 16, 16)] * s
            return c2
        lax.fori_loop(0, K // 16, _srow, 0)

        pltpu.sync_copy(exb, dnacc.at[db], add=True)
        pltpu.sync_copy(rows, accum.at[db], add=True)
        return c
    lax.fori_loop(0, NB, _blk, 0)

    plsc.subcore_barrier()

    def _copy_out(p, dn):
        def _co(i, c):
            pltpu.sync_copy(accum.at[pl.ds(r0 + i * K, K)], rows)
            pltpu.sync_copy(rows, p.at[pl.ds(r0 + i * K, K)])
            return c
        lax.fori_loop(0, nfull, _co, 0)
        pltpu.sync_copy(accum.at[pl.ds(r0 + nfull * K, rem)], rows.at[pl.ds(0, rem)])
        pltpu.sync_copy(rows.at[pl.ds(0, rem)], p.at[pl.ds(r0 + nfull * K, rem)])

        @pl.when(sid == NS - 1)
        def _():
            pltpu.sync_copy(accum.at[pl.ds(NS * RPT, RTAIL)],
                            rows.at[pl.ds(0, RTAIL)])
            pltpu.sync_copy(rows.at[pl.ds(0, RTAIL)],
                            p.at[pl.ds(NS * RPT, RTAIL)])
        pltpu.sync_copy(dnacc.at[pl.ds(sid * 640, 640)], zb)
        pltpu.sync_copy(zb, dn.at[pl.ds(sid * 640, 640)])

    @pl.when(cid == 0)
    def _():
        _copy_out(p0, dn0)

    @pl.when(cid == 1)
    def _():
        _copy_out(p1, dn1)


def _layer_sc(sidx, didx, h, al2, es):
    p0, p1, dn0, dn1 = _edge_pass(
        sidx, didx, h,
        al2[:, 0], al2[:, 1], es[:, 0])
    d0 = dn0[:N].reshape(N, 1)
    d1 = dn1[:N].reshape(N, 1)
    return p0, p1, d0, d1


def kernel(x, adj_t, W1, a_src1, a_dst1, b1, W2, a_src2, a_dst2, b2,
           W3, a_src3, a_dst3, b3):
    sidx = adj_t[0]
    didx = adj_t[1]
    a21 = jnp.stack([a_src1, a_dst1], axis=1)
    a22 = jnp.stack([a_src2, a_dst2], axis=1)
    a23 = jnp.stack([a_src3, a_dst3], axis=1)

    h1, al21, es1 = _pre(x, W1, a21)
    p0, p1, d0, d1 = _layer_sc(sidx, didx, h1, al21, es1)
    h2, al22, es2 = _mid(p0, p1, d0, d1, h1, b1.reshape(1, F), W2, a22)
    p0, p1, d0, d1 = _layer_sc(sidx, didx, h2, al22, es2)
    h3, al23, es3 = _mid(p0, p1, d0, d1, h2, b2.reshape(1, F), W3, a23)
    p0, p1, d0, d1 = _layer_sc(sidx, didx, h3, al23, es3)
    return _fin(p0, p1, d0, d1, h3, b3.reshape(1, F))


# speedup vs baseline: 32.9938x; 1.2550x over previous
import functools

import jax
import jax.numpy as jnp
from jax import lax
from jax.experimental import pallas as pl
from jax.experimental.pallas import tpu as pltpu
from jax.experimental.pallas import tpu_sc as plsc

N = 10000
EDGES = 320000
F = 128

NC, NS = 2, 16
NW = NC * NS
EW = EDGES // NW
K = 64
NBF = EW // K
REM = EW - NBF * K
RPT = 624
RTAIL = N - NS * RPT
DPAD = 640 * NS

BN = 1000


def _pre_body(x_ref, w_ref, a2_ref, h_ref, al2_ref, es_ref):
    h = jnp.dot(x_ref[...], w_ref[...], preferred_element_type=jnp.float32)
    h_ref[...] = h
    al2 = jnp.dot(h, a2_ref[...], preferred_element_type=jnp.float32)
    al2_ref[...] = al2
    e = al2[:, :1] + al2[:, 1:2]
    es_ref[...] = jnp.where(e >= 0, e, 0.2 * e)


def _pre(x, W, a2):
    return pl.pallas_call(
        _pre_body,
        grid=(N // BN,),
        in_specs=[pl.BlockSpec((BN, F), lambda i: (i, 0)),
                  pl.BlockSpec((F, F), lambda i: (0, 0)),
                  pl.BlockSpec((F, 2), lambda i: (0, 0))],
        out_specs=[pl.BlockSpec((BN, F), lambda i: (i, 0)),
                   pl.BlockSpec((BN, 2), lambda i: (i, 0)),
                   pl.BlockSpec((BN, 1), lambda i: (i, 0))],
        out_shape=[jax.ShapeDtypeStruct((N, F), jnp.float32),
                   jax.ShapeDtypeStruct((N, 2), jnp.float32),
                   jax.ShapeDtypeStruct((N, 1), jnp.float32)],
    )(x, W, a2)


def _mid_body(p0_ref, p1_ref, d0_ref, d1_ref, hp_ref, b_ref, w_ref, a2_ref,
              hn_ref, al2_ref, es_ref):
    acc = p0_ref[...] + p1_ref[...] + hp_ref[...]
    inv = 1.0 / (d0_ref[...] + d1_ref[...] + 1.0 + 1e-16)
    o = acc * inv + b_ref[...]
    act = jnp.where(o > 0, o, jnp.exp(o) - 1.0)
    hn = jnp.dot(act, w_ref[...], preferred_element_type=jnp.float32)
    hn_ref[...] = hn
    al2 = jnp.dot(hn, a2_ref[...], preferred_element_type=jnp.float32)
    al2_ref[...] = al2
    e = al2[:, :1] + al2[:, 1:2]
    es_ref[...] = jnp.where(e >= 0, e, 0.2 * e)


def _mid(p0, p1, d0, d1, hp, b, W, a2):
    return pl.pallas_call(
        _mid_body,
        grid=(N // BN,),
        in_specs=[pl.BlockSpec((BN, F), lambda i: (i, 0)),
                  pl.BlockSpec((BN, F), lambda i: (i, 0)),
                  pl.BlockSpec((BN, 1), lambda i: (i, 0)),
                  pl.BlockSpec((BN, 1), lambda i: (i, 0)),
                  pl.BlockSpec((BN, F), lambda i: (i, 0)),
                  pl.BlockSpec((1, F), lambda i: (0, 0)),
                  pl.BlockSpec((F, F), lambda i: (0, 0)),
                  pl.BlockSpec((F, 2), lambda i: (0, 0))],
        out_specs=[pl.BlockSpec((BN, F), lambda i: (i, 0)),
                   pl.BlockSpec((BN, 2), lambda i: (i, 0)),
                   pl.BlockSpec((BN, 1), lambda i: (i, 0))],
        out_shape=[jax.ShapeDtypeStruct((N, F), jnp.float32),
                   jax.ShapeDtypeStruct((N, 2), jnp.float32),
                   jax.ShapeDtypeStruct((N, 1), jnp.float32)],
    )(p0, p1, d0, d1, hp, b, W, a2)


def _fin_body(p0_ref, p1_ref, d0_ref, d1_ref, hp_ref, b_ref, out_ref):
    acc = p0_ref[...] + p1_ref[...] + hp_ref[...]
    inv = 1.0 / (d0_ref[...] + d1_ref[...] + 1.0 + 1e-16)
    o = acc * inv + b_ref[...]
    m = jnp.max(o, axis=-1, keepdims=True)
    z = o - m
    out_ref[...] = z - jnp.log(jnp.sum(jnp.exp(z), axis=-1, keepdims=True))


def _fin(p0, p1, d0, d1, hp, b):
    return pl.pallas_call(
        _fin_body,
        grid=(N // BN,),
        in_specs=[pl.BlockSpec((BN, F), lambda i: (i, 0)),
                  pl.BlockSpec((BN, F), lambda i: (i, 0)),
                  pl.BlockSpec((BN, 1), lambda i: (i, 0)),
                  pl.BlockSpec((BN, 1), lambda i: (i, 0)),
                  pl.BlockSpec((BN, F), lambda i: (i, 0)),
                  pl.BlockSpec((1, F), lambda i: (0, 0))],
        out_specs=pl.BlockSpec((BN, F), lambda i: (i, 0)),
        out_shape=jax.ShapeDtypeStruct((N, F), jnp.float32),
    )(p0, p1, d0, d1, hp, b)


_MESH = plsc.VectorSubcoreMesh(core_axis_name="c", subcore_axis_name="s")


@functools.partial(
    pl.kernel,
    out_type=(jax.ShapeDtypeStruct((N, F), jnp.float32),
              jax.ShapeDtypeStruct((N, F), jnp.float32),
              jax.ShapeDtypeStruct((DPAD,), jnp.float32),
              jax.ShapeDtypeStruct((DPAD,), jnp.float32)),
    mesh=_MESH,
    compiler_params=pltpu.CompilerParams(needs_layout_passes=False),
    scratch_types=[
        pltpu.VMEM_SHARED((N, F), jnp.float32),
        pltpu.VMEM_SHARED((DPAD,), jnp.float32),
        pltpu.VMEM((N,), jnp.float32),
        pltpu.VMEM((N,), jnp.float32),
        pltpu.VMEM((N,), jnp.float32),
        pltpu.VMEM((K,), jnp.int32),
        pltpu.VMEM((K,), jnp.int32),
        pltpu.VMEM((K,), jnp.int32),
        pltpu.VMEM((K,), jnp.int32),
        pltpu.VMEM((K,), jnp.float32),
        pltpu.VMEM((K,), jnp.float32),
        pltpu.VMEM((K, F), jnp.float32),
        pltpu.VMEM((K, F), jnp.float32),
        pltpu.VMEM((REM,), jnp.int32),
        pltpu.VMEM((REM,), jnp.int32),
        pltpu.VMEM((REM,), jnp.float32),
        pltpu.VMEM((640,), jnp.float32),
        pltpu.SemaphoreType.DMA,
        pltpu.SemaphoreType.DMA,
    ],
)
def _edge_pass(sidx, didx, h, als, ald, esv,
               p0, p1, dn0, dn1,
               accum, dnacc, as_l, ad_l, es_l,
               sb0, sb1, db0, db1, exb0, exb1, rows0, rows1,
               sbr, dbr, exbr, zb, semg0, semg1):
    cid = lax.axis_index("c")
    sid = lax.axis_index("s")
    wid = cid * NS + sid

    pltpu.sync_copy(als, as_l)
    pltpu.sync_copy(ald, ad_l)
    pltpu.sync_copy(esv, es_l)

    z16 = jnp.zeros((16,), jnp.float32)

    def _z1(i, c):
        zb[pl.ds(i * 16, 16)] = z16
        return c
    lax.fori_loop(0, 640 // 16, _z1, 0)

    def _zr(k, c):
        for g in range(F // 16):
            rows0[k, pl.ds(g * 16, 16)] = z16
        return c
    lax.fori_loop(0, K, _zr, 0)

    pltpu.sync_copy(zb, dnacc.at[pl.ds(sid * 640, 640)])
    r0 = sid * RPT
    nfull = RPT // K
    rem = RPT - nfull * K

    def _za(i, c):
        pltpu.sync_copy(rows0, accum.at[pl.ds(r0 + i * K, K)])
        return c
    lax.fori_loop(0, nfull, _za, 0)
    pltpu.sync_copy(rows0.at[pl.ds(0, rem)], accum.at[pl.ds(r0 + nfull * K, rem)])

    @pl.when(sid == NS - 1)
    def _():
        pltpu.sync_copy(rows0.at[pl.ds(0, RTAIL)],
                        accum.at[pl.ds(NS * RPT, RTAIL)])
    plsc.subcore_barrier()

    base = wid * EW
    sbs, dbs = (sb0, sb1), (db0, db1)
    exbs, rowss, semgs = (exb0, exb1), (rows0, rows1), (semg0, semg1)

    def _load_and_fire(i, b):
        off = base + i * K
        pltpu.sync_copy(sidx.at[pl.ds(off, K)], sbs[b])
        pltpu.sync_copy(didx.at[pl.ds(off, K)], dbs[b])
        pltpu.async_copy(h.at[sbs[b]], rowss[b], semgs[b])

    def _scalar_pass(sb, db, exb):
        def _grp(j, c2):
            s16 = sb[pl.ds(j * 16, 16)]
            d16 = db[pl.ds(j * 16, 16)]
            a = plsc.load_gather(as_l, [s16]) + plsc.load_gather(ad_l, [d16])
            e = jnp.where(a >= 0, a, 0.2 * a)
            exb[pl.ds(j * 16, 16)] = jnp.exp(e - plsc.load_gather(es_l, [d16]))
            return c2
        lax.fori_loop(0, K // 16, _grp, 0)

    def _scale(exb, rows):
        def _srow(j, c2):
            ex16 = exb[pl.ds(j * 16, 16)]
            bk = j * 16
            for l in range(16):
                s = ex16[l]
                for g in range(F // 16):
                    rows[bk + l, pl.ds(g * 16, 16)] = rows[bk + l, pl.ds(g * 16, 16)] * s
            return c2
        lax.fori_loop(0, K // 16, _srow, 0)

    _load_and_fire(0, 0)

    def _pair(i2, c):
        for b in range(2):
            i = i2 * 2 + b
            nb = 1 - b

            @pl.when(i < NBF - 1)
            def _():
                _load_and_fire(i + 1, nb)
            _scalar_pass(sbs[b], dbs[b], exbs[b])
            pltpu.make_async_copy(h.at[sbs[b]], rowss[b], semgs[b]).wait()
            _scale(exbs[b], rowss[b])
            pltpu.sync_copy(exbs[b], dnacc.at[dbs[b]], add=True)
            pltpu.sync_copy(rowss[b], accum.at[dbs[b]], add=True)
        return c
    lax.fori_loop(0, NBF // 2, _pair, 0)

    offr = base + NBF * K
    pltpu.sync_copy(sidx.at[pl.ds(offr, REM)], sbr)
    pltpu.sync_copy(didx.at[pl.ds(offr, REM)], dbr)
    cp = pltpu.async_copy(h.at[sbr], rows0.at[pl.ds(0, REM)], semg0)
    s16 = sbr[...]
    d16 = dbr[...]
    a = plsc.load_gather(as_l, [s16]) + plsc.load_gather(ad_l, [d16])
    e = jnp.where(a >= 0, a, 0.2 * a)
    exr = jnp.exp(e - plsc.load_gather(es_l, [d16]))
    exbr[...] = exr
    cp.wait()
    for l in range(REM):
        s = exr[l]
        for g in range(F // 16):
            rows0[l, pl.ds(g * 16, 16)] = rows0[l, pl.ds(g * 16, 16)] * s
    pltpu.sync_copy(exbr, dnacc.at[dbr], add=True)
    pltpu.sync_copy(rows0.at[pl.ds(0, REM)], accum.at[dbr], add=True)

    plsc.subcore_barrier()

    def _copy_out(p, dn):
        def _co(i, c):
            pltpu.sync_copy(accum.at[pl.ds(r0 + i * K, K)], rows0)
            pltpu.sync_copy(rows0, p.at[pl.ds(r0 + i * K, K)])
            return c
        lax.fori_loop(0, nfull, _co, 0)
        pltpu.sync_copy(accum.at[pl.ds(r0 + nfull * K, rem)], rows0.at[pl.ds(0, rem)])
        pltpu.sync_copy(rows0.at[pl.ds(0, rem)], p.at[pl.ds(r0 + nfull * K, rem)])

        @pl.when(sid == NS - 1)
        def _():
            pltpu.sync_copy(accum.at[pl.ds(NS * RPT, RTAIL)],
                            rows0.at[pl.ds(0, RTAIL)])
            pltpu.sync_copy(rows0.at[pl.ds(0, RTAIL)],
                            p.at[pl.ds(NS * RPT, RTAIL)])
        pltpu.sync_copy(dnacc.at[pl.ds(sid * 640, 640)], zb)
        pltpu.sync_copy(zb, dn.at[pl.ds(sid * 640, 640)])

    @pl.when(cid == 0)
    def _():
        _copy_out(p0, dn0)

    @pl.when(cid == 1)
    def _():
        _copy_out(p1, dn1)


def _layer_sc(sidx, didx, h, al2, es):
    p0, p1, dn0, dn1 = _edge_pass(
        sidx, didx, h,
        al2[:, 0], al2[:, 1], es[:, 0])
    d0 = dn0[:N].reshape(N, 1)
    d1 = dn1[:N].reshape(N, 1)
    return p0, p1, d0, d1


def kernel(x, adj_t, W1, a_src1, a_dst1, b1, W2, a_src2, a_dst2, b2,
           W3, a_src3, a_dst3, b3):
    sidx = adj_t[0]
    didx = adj_t[1]
    a21 = jnp.stack([a_src1, a_dst1], axis=1)
    a22 = jnp.stack([a_src2, a_dst2], axis=1)
    a23 = jnp.stack([a_src3, a_dst3], axis=1)

    h1, al21, es1 = _pre(x, W1, a21)
    p0, p1, d0, d1 = _layer_sc(sidx, didx, h1, al21, es1)
    h2, al22, es2 = _mid(p0, p1, d0, d1, h1, b1.reshape(1, F), W2, a22)
    p0, p1, d0, d1 = _layer_sc(sidx, didx, h2, al22, es2)
    h3, al23, es3 = _mid(p0, p1, d0, d1, h2, b2.reshape(1, F), W3, a23)
    p0, p1, d0, d1 = _layer_sc(sidx, didx, h3, al23, es3)
    return _fin(p0, p1, d0, d1, h3, b3.reshape(1, F))


# speedup vs baseline: 41.3671x; 1.2538x over previous
import functools

import jax
import jax.numpy as jnp
from jax import lax
from jax.experimental import pallas as pl
from jax.experimental.pallas import tpu as pltpu
from jax.experimental.pallas import tpu_sc as plsc

N = 10000
EDGES = 320000
F = 128

NC, NS = 2, 16
NW = NC * NS
EW = EDGES // NW
K = 64
NBF = EW // K
REM = EW - NBF * K
RPT = 624
RTAIL = N - NS * RPT

BN = 1000


def _pre_body(x_ref, w_ref, a2_ref, h_ref, al2_ref):
    h = jnp.dot(x_ref[...], w_ref[...], preferred_element_type=jnp.float32)
    h_ref[...] = h
    al2_ref[...] = jnp.dot(h, a2_ref[...], preferred_element_type=jnp.float32)


def _pre(x, W, a2):
    return pl.pallas_call(
        _pre_body,
        grid=(N // BN,),
        in_specs=[pl.BlockSpec((BN, F), lambda i: (i, 0)),
                  pl.BlockSpec((F, F), lambda i: (0, 0)),
                  pl.BlockSpec((F, 2), lambda i: (0, 0))],
        out_specs=[pl.BlockSpec((BN, F), lambda i: (i, 0)),
                   pl.BlockSpec((BN, 2), lambda i: (i, 0))],
        out_shape=[jax.ShapeDtypeStruct((N, F), jnp.float32),
                   jax.ShapeDtypeStruct((N, 2), jnp.float32)],
    )(x, W, a2)


def _mid_body(p0_ref, p1_ref, d0_ref, d1_ref, hp_ref, b_ref, w_ref, a2_ref,
              hn_ref, al2_ref):
    acc = p0_ref[...] + p1_ref[...] + hp_ref[...]
    inv = 1.0 / (d0_ref[...] + d1_ref[...] + 1.0 + 1e-16)
    o = acc * inv + b_ref[...]
    act = jnp.where(o > 0, o, jnp.exp(o) - 1.0)
    hn = jnp.dot(act, w_ref[...], preferred_element_type=jnp.float32)
    hn_ref[...] = hn
    al2_ref[...] = jnp.dot(hn, a2_ref[...], preferred_element_type=jnp.float32)


def _mid(p0, p1, d0, d1, hp, b, W, a2):
    return pl.pallas_call(
        _mid_body,
        grid=(N // BN,),
        in_specs=[pl.BlockSpec((BN, F), lambda i: (i, 0)),
                  pl.BlockSpec((BN, F), lambda i: (i, 0)),
                  pl.BlockSpec((BN, 1), lambda i: (i, 0)),
                  pl.BlockSpec((BN, 1), lambda i: (i, 0)),
                  pl.BlockSpec((BN, F), lambda i: (i, 0)),
                  pl.BlockSpec((1, F), lambda i: (0, 0)),
                  pl.BlockSpec((F, F), lambda i: (0, 0)),
                  pl.BlockSpec((F, 2), lambda i: (0, 0))],
        out_specs=[pl.BlockSpec((BN, F), lambda i: (i, 0)),
                   pl.BlockSpec((BN, 2), lambda i: (i, 0))],
        out_shape=[jax.ShapeDtypeStruct((N, F), jnp.float32),
                   jax.ShapeDtypeStruct((N, 2), jnp.float32)],
    )(p0, p1, d0, d1, hp, b, W, a2)


def _fin_body(p0_ref, p1_ref, d0_ref, d1_ref, hp_ref, b_ref, out_ref):
    acc = p0_ref[...] + p1_ref[...] + hp_ref[...]
    inv = 1.0 / (d0_ref[...] + d1_ref[...] + 1.0 + 1e-16)
    o = acc * inv + b_ref[...]
    m = jnp.max(o, axis=-1, keepdims=True)
    z = o - m
    out_ref[...] = z - jnp.log(jnp.sum(jnp.exp(z), axis=-1, keepdims=True))


def _fin(p0, p1, d0, d1, hp, b):
    return pl.pallas_call(
        _fin_body,
        grid=(N // BN,),
        in_specs=[pl.BlockSpec((BN, F), lambda i: (i, 0)),
                  pl.BlockSpec((BN, F), lambda i: (i, 0)),
                  pl.BlockSpec((BN, 1), lambda i: (i, 0)),
                  pl.BlockSpec((BN, 1), lambda i: (i, 0)),
                  pl.BlockSpec((BN, F), lambda i: (i, 0)),
                  pl.BlockSpec((1, F), lambda i: (0, 0))],
        out_specs=pl.BlockSpec((BN, F), lambda i: (i, 0)),
        out_shape=jax.ShapeDtypeStruct((N, F), jnp.float32),
    )(p0, p1, d0, d1, hp, b)


_MESH = plsc.VectorSubcoreMesh(core_axis_name="c", subcore_axis_name="s")


@functools.partial(
    pl.kernel,
    out_type=(jax.ShapeDtypeStruct((N, F), jnp.float32),
              jax.ShapeDtypeStruct((N, F), jnp.float32),
              jax.ShapeDtypeStruct((640 * NS,), jnp.float32),
              jax.ShapeDtypeStruct((640 * NS,), jnp.float32)),
    mesh=_MESH,
    compiler_params=pltpu.CompilerParams(needs_layout_passes=False),
    scratch_types=[
        pltpu.VMEM_SHARED((N, F), jnp.float32),
        pltpu.VMEM_SHARED((640 * NS,), jnp.float32),
        pltpu.VMEM((N,), jnp.float32),
        pltpu.VMEM((N,), jnp.float32),
        pltpu.VMEM((3, K), jnp.int32),
        pltpu.VMEM((3, K), jnp.int32),
        pltpu.VMEM((3, K), jnp.float32),
        pltpu.VMEM((K, F), jnp.float32),
        pltpu.VMEM((K, F), jnp.float32),
        pltpu.VMEM((K, F), jnp.float32),
        pltpu.VMEM((REM,), jnp.int32),
        pltpu.VMEM((REM,), jnp.int32),
        pltpu.VMEM((REM,), jnp.float32),
        pltpu.VMEM((640,), jnp.float32),
        pltpu.SemaphoreType.DMA,
        pltpu.SemaphoreType.DMA,
        pltpu.SemaphoreType.DMA,
        pltpu.SemaphoreType.DMA,
        pltpu.SemaphoreType.DMA,
        pltpu.SemaphoreType.DMA,
    ],
)
def _edge_pass(sidx, didx, h, als, ald,
               p0, p1, dn0, dn1,
               accum, dnacc, as_l, ad_l, sring, dring, exring,
               rows0, rows1, rows2, sbr, dbr, exbr, zb,
               semg0, semg1, semg2, sems0, sems1, sems2):
    cid = lax.axis_index("c")
    sid = lax.axis_index("s")
    wid = cid * NS + sid

    pltpu.sync_copy(als, as_l)
    pltpu.sync_copy(ald, ad_l)

    z16 = jnp.zeros((16,), jnp.float32)

    def _z1(i, c):
        zb[pl.ds(i * 16, 16)] = z16
        return c
    lax.fori_loop(0, 640 // 16, _z1, 0)

    def _zr(k, c):
        for g in range(F // 16):
            rows0[k, pl.ds(g * 16, 16)] = z16
        return c
    lax.fori_loop(0, K, _zr, 0)

    pltpu.sync_copy(zb, dnacc.at[pl.ds(sid * 640, 640)])
    r0 = sid * RPT
    nfull = RPT // K
    rem = RPT - nfull * K

    def _za(i, c):
        pltpu.sync_copy(rows0, accum.at[pl.ds(r0 + i * K, K)])
        return c
    lax.fori_loop(0, nfull, _za, 0)
    pltpu.sync_copy(rows0.at[pl.ds(0, rem)], accum.at[pl.ds(r0 + nfull * K, rem)])

    @pl.when(sid == NS - 1)
    def _():
        pltpu.sync_copy(rows0.at[pl.ds(0, RTAIL)],
                        accum.at[pl.ds(NS * RPT, RTAIL)])
    plsc.subcore_barrier()

    base = wid * EW
    rowss = (rows0, rows1, rows2)
    semgs = (semg0, semg1, semg2)
    semss = (sems0, sems1, sems2)

    def _load_and_fire(i, b):
        off = base + i * K
        pltpu.sync_copy(sidx.at[pl.ds(off, K)], sring.at[b])
        pltpu.sync_copy(didx.at[pl.ds(off, K)], dring.at[b])
        pltpu.async_copy(h.at[sring.at[b]], rowss[b], semgs[b])

    def _drain_scatter(b):
        pltpu.make_async_copy(exring.at[b], dnacc.at[dring.at[b]], semss[b]).wait()
        pltpu.make_async_copy(rowss[b], accum.at[dring.at[b]], semss[b]).wait()

    def _scalar_pass(b):
        def _grp(j, c2):
            s16 = sring[b, pl.ds(j * 16, 16)]
            d16 = dring[b, pl.ds(j * 16, 16)]
            a = plsc.load_gather(as_l, [s16]) + plsc.load_gather(ad_l, [d16])
            sl = plsc.load_gather(as_l, [d16]) + plsc.load_gather(ad_l, [d16])
            e = jnp.where(a >= 0, a, 0.2 * a)
            es = jnp.where(sl >= 0, sl, 0.2 * sl)
            exring[b, pl.ds(j * 16, 16)] = jnp.exp(e - es)
            return c2
        lax.fori_loop(0, K // 16, _grp, 0)

    def _scale(b):
        rows = rowss[b]

        def _srow(j, c2):
            ex16 = exring[b, pl.ds(j * 16, 16)]
            bk = j * 16
            for l in range(16):
                s = ex16[l]
                for g in range(F // 16):
                    rows[bk + l, pl.ds(g * 16, 16)] = rows[bk + l, pl.ds(g * 16, 16)] * s
            return c2
        lax.fori_loop(0, K // 16, _srow, 0)

    _load_and_fire(0, 0)

    def _tri(i3, c):
        for b in range(3):
            i = i3 * 3 + b
            nb = (b + 1) % 3

            @pl.when((i >= 2) & (i < NBF - 1))
            def _():
                _drain_scatter(nb)
                _load_and_fire(i + 1, nb)

            @pl.when((i >= 2) & (i >= NBF - 1))
            def _():
                _drain_scatter(nb)

            @pl.when((i < 2) & (i < NBF - 1))
            def _():
                _load_and_fire(i + 1, nb)

            _scalar_pass(b)
            pltpu.make_async_copy(h.at[sring.at[b]], rowss[b], semgs[b]).wait()
            _scale(b)
            pltpu.async_copy(exring.at[b], dnacc.at[dring.at[b]], semss[b], add=True)
            pltpu.async_copy(rowss[b], accum.at[dring.at[b]], semss[b], add=True)
        return c
    lax.fori_loop(0, NBF // 3, _tri, 0)

    _drain_scatter((NBF - 2) % 3)
    _drain_scatter((NBF - 1) % 3)

    offr = base + NBF * K
    pltpu.sync_copy(sidx.at[pl.ds(offr, REM)], sbr)
    pltpu.sync_copy(didx.at[pl.ds(offr, REM)], dbr)
    cp = pltpu.async_copy(h.at[sbr], rows0.at[pl.ds(0, REM)], semg0)
    s16 = sbr[...]
    d16 = dbr[...]
    a = plsc.load_gather(as_l, [s16]) + plsc.load_gather(ad_l, [d16])
    sl = plsc.load_gather(as_l, [d16]) + plsc.load_gather(ad_l, [d16])
    e = jnp.where(a >= 0, a, 0.2 * a)
    es = jnp.where(sl >= 0, sl, 0.2 * sl)
    exr = jnp.exp(e - es)
    exbr[...] = exr
    cp.wait()
    for l in range(REM):
        s = exr[l]
        for g in range(F // 16):
            rows0[l, pl.ds(g * 16, 16)] = rows0[l, pl.ds(g * 16, 16)] * s
    pltpu.sync_copy(exbr, dnacc.at[dbr], add=True)
    pltpu.sync_copy(rows0.at[pl.ds(0, REM)], accum.at[dbr], add=True)

    plsc.subcore_barrier()

    def _copy_out(p, dn):
        def _co(i, c):
            pltpu.sync_copy(accum.at[pl.ds(r0 + i * K, K)], rows0)
            pltpu.sync_copy(rows0, p.at[pl.ds(r0 + i * K, K)])
            return c
        lax.fori_loop(0, nfull, _co, 0)
        pltpu.sync_copy(accum.at[pl.ds(r0 + nfull * K, rem)], rows0.at[pl.ds(0, rem)])
        pltpu.sync_copy(rows0.at[pl.ds(0, rem)], p.at[pl.ds(r0 + nfull * K, rem)])

        @pl.when(sid == NS - 1)
        def _():
            pltpu.sync_copy(accum.at[pl.ds(NS * RPT, RTAIL)],
                            rows0.at[pl.ds(0, RTAIL)])
            pltpu.sync_copy(rows0.at[pl.ds(0, RTAIL)],
                            p.at[pl.ds(NS * RPT, RTAIL)])
        pltpu.sync_copy(dnacc.at[pl.ds(sid * 640, 640)], zb)
        pltpu.sync_copy(zb, dn.at[pl.ds(sid * 640, 640)])

    @pl.when(cid == 0)
    def _():
        _copy_out(p0, dn0)

    @pl.when(cid == 1)
    def _():
        _copy_out(p1, dn1)


def _layer_sc(sidx, didx, h, al2):
    p0, p1, dn0, dn1 = _edge_pass(sidx, didx, h, al2[:, 0], al2[:, 1])
    d0 = dn0[:N].reshape(N, 1)
    d1 = dn1[:N].reshape(N, 1)
    return p0, p1, d0, d1


def kernel(x, adj_t, W1, a_src1, a_dst1, b1, W2, a_src2, a_dst2, b2,
           W3, a_src3, a_dst3, b3):
    sidx = adj_t[0]
    didx = adj_t[1]
    a21 = jnp.stack([a_src1, a_dst1], axis=1)
    a22 = jnp.stack([a_src2, a_dst2], axis=1)
    a23 = jnp.stack([a_src3, a_dst3], axis=1)

    h1, al21 = _pre(x, W1, a21)
    p0, p1, d0, d1 = _layer_sc(sidx, didx, h1, al21)
    h2, al22 = _mid(p0, p1, d0, d1, h1, b1.reshape(1, F), W2, a22)
    p0, p1, d0, d1 = _layer_sc(sidx, didx, h2, al22)
    h3, al23 = _mid(p0, p1, d0, d1, h2, b2.reshape(1, F), W3, a23)
    p0, p1, d0, d1 = _layer_sc(sidx, didx, h3, al23)
    return _fin(p0, p1, d0, d1, h3, b3.reshape(1, F))


# speedup vs baseline: 51.9732x; 1.2564x over previous
import functools

import jax
import jax.numpy as jnp
from jax import lax
from jax.experimental import pallas as pl
from jax.experimental.pallas import tpu as pltpu
from jax.experimental.pallas import tpu_sc as plsc

N = 10000
EDGES = 320000
F = 128

NC, NS = 2, 16
NW = NC * NS
EW = EDGES // NW
K = 64
NBF = EW // K
REM = EW - NBF * K
RPT = 624
RTAIL = N - NS * RPT

BN = 1000


def _pre_body(x_ref, w_ref, a2_ref, h_ref, al2_ref):
    h = jnp.dot(x_ref[...], w_ref[...], preferred_element_type=jnp.float32)
    h_ref[...] = h
    al2_ref[...] = jnp.dot(h, a2_ref[...], preferred_element_type=jnp.float32)


def _pre(x, W, a2):
    return pl.pallas_call(
        _pre_body,
        grid=(N // BN,),
        in_specs=[pl.BlockSpec((BN, F), lambda i: (i, 0)),
                  pl.BlockSpec((F, F), lambda i: (0, 0)),
                  pl.BlockSpec((F, 2), lambda i: (0, 0))],
        out_specs=[pl.BlockSpec((BN, F), lambda i: (i, 0)),
                   pl.BlockSpec((BN, 2), lambda i: (i, 0))],
        out_shape=[jax.ShapeDtypeStruct((N, F), jnp.float32),
                   jax.ShapeDtypeStruct((N, 2), jnp.float32)],
    )(x, W, a2)


def _mid_body(p0_ref, p1_ref, d0_ref, d1_ref, hp_ref, b_ref, w_ref, a2_ref,
              hn_ref, al2_ref):
    acc = p0_ref[...] + p1_ref[...] + hp_ref[...]
    inv = 1.0 / (d0_ref[...] + d1_ref[...] + 1.0 + 1e-16)
    o = acc * inv + b_ref[...]
    act = jnp.where(o > 0, o, jnp.exp(o) - 1.0)
    hn = jnp.dot(act, w_ref[...], preferred_element_type=jnp.float32)
    hn_ref[...] = hn
    al2_ref[...] = jnp.dot(hn, a2_ref[...], preferred_element_type=jnp.float32)


def _mid(p0, p1, d0, d1, hp, b, W, a2):
    return pl.pallas_call(
        _mid_body,
        grid=(N // BN,),
        in_specs=[pl.BlockSpec((BN, F), lambda i: (i, 0)),
                  pl.BlockSpec((BN, F), lambda i: (i, 0)),
                  pl.BlockSpec((BN, 1), lambda i: (i, 0)),
                  pl.BlockSpec((BN, 1), lambda i: (i, 0)),
                  pl.BlockSpec((BN, F), lambda i: (i, 0)),
                  pl.BlockSpec((1, F), lambda i: (0, 0)),
                  pl.BlockSpec((F, F), lambda i: (0, 0)),
                  pl.BlockSpec((F, 2), lambda i: (0, 0))],
        out_specs=[pl.BlockSpec((BN, F), lambda i: (i, 0)),
                   pl.BlockSpec((BN, 2), lambda i: (i, 0))],
        out_shape=[jax.ShapeDtypeStruct((N, F), jnp.float32),
                   jax.ShapeDtypeStruct((N, 2), jnp.float32)],
    )(p0, p1, d0, d1, hp, b, W, a2)


def _fin_body(p0_ref, p1_ref, d0_ref, d1_ref, hp_ref, b_ref, out_ref):
    acc = p0_ref[...] + p1_ref[...] + hp_ref[...]
    inv = 1.0 / (d0_ref[...] + d1_ref[...] + 1.0 + 1e-16)
    o = acc * inv + b_ref[...]
    m = jnp.max(o, axis=-1, keepdims=True)
    z = o - m
    out_ref[...] = z - jnp.log(jnp.sum(jnp.exp(z), axis=-1, keepdims=True))


def _fin(p0, p1, d0, d1, hp, b):
    return pl.pallas_call(
        _fin_body,
        grid=(N // BN,),
        in_specs=[pl.BlockSpec((BN, F), lambda i: (i, 0)),
                  pl.BlockSpec((BN, F), lambda i: (i, 0)),
                  pl.BlockSpec((BN, 1), lambda i: (i, 0)),
                  pl.BlockSpec((BN, 1), lambda i: (i, 0)),
                  pl.BlockSpec((BN, F), lambda i: (i, 0)),
                  pl.BlockSpec((1, F), lambda i: (0, 0))],
        out_specs=pl.BlockSpec((BN, F), lambda i: (i, 0)),
        out_shape=jax.ShapeDtypeStruct((N, F), jnp.float32),
    )(p0, p1, d0, d1, hp, b)


_MESH = plsc.VectorSubcoreMesh(core_axis_name="c", subcore_axis_name="s")


@functools.partial(
    pl.kernel,
    out_type=(jax.ShapeDtypeStruct((N, F), jnp.float32),
              jax.ShapeDtypeStruct((N, F), jnp.float32),
              jax.ShapeDtypeStruct((640 * NS,), jnp.float32),
              jax.ShapeDtypeStruct((640 * NS,), jnp.float32)),
    mesh=_MESH,
    compiler_params=pltpu.CompilerParams(needs_layout_passes=False),
    scratch_types=[
        pltpu.VMEM_SHARED((N, F), jnp.float32),
        pltpu.VMEM_SHARED((640 * NS,), jnp.float32),
        pltpu.VMEM((N,), jnp.float32),
        pltpu.VMEM((N,), jnp.float32),
        pltpu.VMEM((3, K), jnp.int32),
        pltpu.VMEM((3, K), jnp.int32),
        pltpu.VMEM((3, K), jnp.float32),
        pltpu.VMEM((K, F), jnp.float32),
        pltpu.VMEM((K, F), jnp.float32),
        pltpu.VMEM((K, F), jnp.float32),
        pltpu.VMEM((REM,), jnp.int32),
        pltpu.VMEM((REM,), jnp.int32),
        pltpu.VMEM((REM,), jnp.float32),
        pltpu.VMEM((640,), jnp.float32),
        pltpu.SemaphoreType.DMA,
        pltpu.SemaphoreType.DMA,
        pltpu.SemaphoreType.DMA,
        pltpu.SemaphoreType.DMA,
        pltpu.SemaphoreType.DMA,
        pltpu.SemaphoreType.DMA,
        pltpu.SemaphoreType.DMA,
        pltpu.SemaphoreType.DMA,
        pltpu.SemaphoreType.DMA,
    ],
)
def _edge_pass(sidx, didx, h, als, ald,
               p0, p1, dn0, dn1,
               accum, dnacc, as_l, ad_l, sring, dring, exring,
               rows0, rows1, rows2, sbr, dbr, exbr, zb,
               semg0, semg1, semg2, sems0, sems1, sems2,
               semi0, semi1, semi2):
    cid = lax.axis_index("c")
    sid = lax.axis_index("s")
    wid = cid * NS + sid

    pltpu.sync_copy(als, as_l)
    pltpu.sync_copy(ald, ad_l)

    z16 = jnp.zeros((16,), jnp.float32)

    def _z1(i, c):
        zb[pl.ds(i * 16, 16)] = z16
        return c
    lax.fori_loop(0, 640 // 16, _z1, 0)

    def _zr(k, c):
        for g in range(F // 16):
            rows0[k, pl.ds(g * 16, 16)] = z16
        return c
    lax.fori_loop(0, K, _zr, 0)

    pltpu.sync_copy(zb, dnacc.at[pl.ds(sid * 640, 640)])
    r0 = sid * RPT
    nfull = RPT // K
    rem = RPT - nfull * K

    def _za(i, c):
        pltpu.sync_copy(rows0, accum.at[pl.ds(r0 + i * K, K)])
        return c
    lax.fori_loop(0, nfull, _za, 0)
    pltpu.sync_copy(rows0.at[pl.ds(0, rem)], accum.at[pl.ds(r0 + nfull * K, rem)])

    @pl.when(sid == NS - 1)
    def _():
        pltpu.sync_copy(rows0.at[pl.ds(0, RTAIL)],
                        accum.at[pl.ds(NS * RPT, RTAIL)])
    plsc.subcore_barrier()

    base = wid * EW
    rowss = (rows0, rows1, rows2)
    semgs = (semg0, semg1, semg2)
    semss = (sems0, sems1, sems2)
    semis = (semi0, semi1, semi2)

    def _load_idx(i, b):
        off = base + i * K
        pltpu.async_copy(sidx.at[pl.ds(off, K)], sring.at[b], semis[b])
        pltpu.async_copy(didx.at[pl.ds(off, K)], dring.at[b], semis[b])

    def _wait_idx_fire_gather(i, b):
        off = base + i * K
        pltpu.make_async_copy(sidx.at[pl.ds(off, K)], sring.at[b], semis[b]).wait()
        pltpu.make_async_copy(didx.at[pl.ds(off, K)], dring.at[b], semis[b]).wait()
        pltpu.async_copy(h.at[sring.at[b]], rowss[b], semgs[b])

    def _load_and_fire(i, b):
        _load_idx(i, b)
        _wait_idx_fire_gather(i, b)

    def _drain_scatter(b):
        pltpu.make_async_copy(exring.at[b], dnacc.at[dring.at[b]], semss[b]).wait()
        pltpu.make_async_copy(rowss[b], accum.at[dring.at[b]], semss[b]).wait()

    def _scalar_pass(b):
        def _grp(j, c2):
            s16 = sring[b, pl.ds(j * 16, 16)]
            d16 = dring[b, pl.ds(j * 16, 16)]
            a = plsc.load_gather(as_l, [s16]) + plsc.load_gather(ad_l, [d16])
            sl = plsc.load_gather(as_l, [d16]) + plsc.load_gather(ad_l, [d16])
            e = jnp.where(a >= 0, a, 0.2 * a)
            es = jnp.where(sl >= 0, sl, 0.2 * sl)
            exring[b, pl.ds(j * 16, 16)] = jnp.exp(e - es)
            return c2
        lax.fori_loop(0, K // 16, _grp, 0)

    def _scale(b):
        rows = rowss[b]

        def _srow(j, c2):
            ex16 = exring[b, pl.ds(j * 16, 16)]
            bk = j * 16
            for l in range(16):
                s = ex16[l]
                for g in range(F // 16):
                    rows[bk + l, pl.ds(g * 16, 16)] = rows[bk + l, pl.ds(g * 16, 16)] * s
            return c2
        lax.fori_loop(0, K // 16, _srow, 0)

    _load_and_fire(0, 0)

    def _tri(i3, c):
        for b in range(3):
            i = i3 * 3 + b
            nb = (b + 1) % 3

            @pl.when(i >= 2)
            def _():
                _drain_scatter(nb)

            @pl.when(i < NBF - 1)
            def _():
                _load_idx(i + 1, nb)

            _scalar_pass(b)

            @pl.when(i < NBF - 1)
            def _():
                _wait_idx_fire_gather(i + 1, nb)

            pltpu.make_async_copy(h.at[sring.at[b]], rowss[b], semgs[b]).wait()
            _scale(b)
            pltpu.async_copy(exring.at[b], dnacc.at[dring.at[b]], semss[b], add=True)
            pltpu.async_copy(rowss[b], accum.at[dring.at[b]], semss[b], add=True)
        return c
    lax.fori_loop(0, NBF // 3, _tri, 0)

    _drain_scatter((NBF - 2) % 3)
    _drain_scatter((NBF - 1) % 3)

    offr = base + NBF * K
    pltpu.sync_copy(sidx.at[pl.ds(offr, REM)], sbr)
    pltpu.sync_copy(didx.at[pl.ds(offr, REM)], dbr)
    cp = pltpu.async_copy(h.at[sbr], rows0.at[pl.ds(0, REM)], semg0)
    s16 = sbr[...]
    d16 = dbr[...]
    a = plsc.load_gather(as_l, [s16]) + plsc.load_gather(ad_l, [d16])
    sl = plsc.load_gather(as_l, [d16]) + plsc.load_gather(ad_l, [d16])
    e = jnp.where(a >= 0, a, 0.2 * a)
    es = jnp.where(sl >= 0, sl, 0.2 * sl)
    exr = jnp.exp(e - es)
    exbr[...] = exr
    cp.wait()
    for l in range(REM):
        s = exr[l]
        for g in range(F // 16):
            rows0[l, pl.ds(g * 16, 16)] = rows0[l, pl.ds(g * 16, 16)] * s
    pltpu.sync_copy(exbr, dnacc.at[dbr], add=True)
    pltpu.sync_copy(rows0.at[pl.ds(0, REM)], accum.at[dbr], add=True)

    plsc.subcore_barrier()

    def _copy_out(p, dn):
        def _co(i, c):
            pltpu.sync_copy(accum.at[pl.ds(r0 + i * K, K)], rows0)
            pltpu.sync_copy(rows0, p.at[pl.ds(r0 + i * K, K)])
            return c
        lax.fori_loop(0, nfull, _co, 0)
        pltpu.sync_copy(accum.at[pl.ds(r0 + nfull * K, rem)], rows0.at[pl.ds(0, rem)])
        pltpu.sync_copy(rows0.at[pl.ds(0, rem)], p.at[pl.ds(r0 + nfull * K, rem)])

        @pl.when(sid == NS - 1)
        def _():
            pltpu.sync_copy(accum.at[pl.ds(NS * RPT, RTAIL)],
                            rows0.at[pl.ds(0, RTAIL)])
            pltpu.sync_copy(rows0.at[pl.ds(0, RTAIL)],
                            p.at[pl.ds(NS * RPT, RTAIL)])
        pltpu.sync_copy(dnacc.at[pl.ds(sid * 640, 640)], zb)
        pltpu.sync_copy(zb, dn.at[pl.ds(sid * 640, 640)])

    @pl.when(cid == 0)
    def _():
        _copy_out(p0, dn0)

    @pl.when(cid == 1)
    def _():
        _copy_out(p1, dn1)


def _layer_sc(sidx, didx, h, al2):
    p0, p1, dn0, dn1 = _edge_pass(sidx, didx, h, al2[:, 0], al2[:, 1])
    d0 = dn0[:N].reshape(N, 1)
    d1 = dn1[:N].reshape(N, 1)
    return p0, p1, d0, d1


def kernel(x, adj_t, W1, a_src1, a_dst1, b1, W2, a_src2, a_dst2, b2,
           W3, a_src3, a_dst3, b3):
    sidx = adj_t[0]
    didx = adj_t[1]
    a21 = jnp.stack([a_src1, a_dst1], axis=1)
    a22 = jnp.stack([a_src2, a_dst2], axis=1)
    a23 = jnp.stack([a_src3, a_dst3], axis=1)

    h1, al21 = _pre(x, W1, a21)
    p0, p1, d0, d1 = _layer_sc(sidx, didx, h1, al21)
    h2, al22 = _mid(p0, p1, d0, d1, h1, b1.reshape(1, F), W2, a22)
    p0, p1, d0, d1 = _layer_sc(sidx, didx, h2, al22)
    h3, al23 = _mid(p0, p1, d0, d1, h2, b2.reshape(1, F), W3, a23)
    p0, p1, d0, d1 = _layer_sc(sidx, didx, h3, al23)
    return _fin(p0, p1, d0, d1, h3, b3.reshape(1, F))
